# baseline (speedup 1.0000x reference)
import functools

import jax
import jax.numpy as jnp
from jax import lax
from jax.experimental import pallas as pl
from jax.experimental.pallas import tpu as pltpu
from jax.experimental.pallas import tpu_sc as plsc

B, N, K, G = 2, 4096, 16, 4
CQ, CV = 256, 128
LPE_OUT = 32
CT = 512
NC, NS = 2, 16
NW = NC * NS
TOT = B * N * K
BGN = B * G * N

BN_BLK = 128
R_BLK = BN_BLK * K


_ROWS_PER_W = TOT // NW
_GCH = 128
_N_GCH = _ROWS_PER_W // _GCH


def _sc_gather_body(table_hbm, idx_hbm, out_hbm, idx_v, rows_v, sem):
    wid = lax.axis_index("s") * NC + lax.axis_index("c")
    base = wid * _ROWS_PER_W

    def chunk(t, carry):
        off = base + t * _GCH
        pltpu.sync_copy(idx_hbm.at[pl.ds(off, _GCH)], idx_v)
        pltpu.async_copy(table_hbm.at[idx_v], rows_v, sem).wait()
        pltpu.sync_copy(rows_v, out_hbm.at[pl.ds(off, _GCH)])
        return carry

    lax.fori_loop(0, _N_GCH, chunk, 0)


def _sc_gather(table, flat_idx):
    call = pl.kernel(
        _sc_gather_body,
        mesh=plsc.VectorSubcoreMesh(core_axis_name="c", subcore_axis_name="s",
                                    num_cores=NC, num_subcores=NS),
        out_type=jax.ShapeDtypeStruct((TOT, CT), jnp.float32),
        scratch_types=[
            pltpu.VMEM((_GCH,), jnp.int32),
            pltpu.VMEM((_GCH, CT), jnp.float32),
            pltpu.SemaphoreType.DMA,
        ],
    )
    return call(table, flat_idx)


_ITEMS = B * N * G * K
_ITEMS_PER_W = _ITEMS // NW
_SCH = 2048
_N_SCH = _ITEMS_PER_W // _SCH


def _sc_scatter_body(pos_hbm, pr_hbm, out_hbm, cbuf, idx_v, p_v):
    wid = lax.axis_index("s") * NC + lax.axis_index("c")
    base = wid * _ITEMS_PER_W

    def zero(i, carry):
        cbuf[pl.ds(i * 16, 16)] = jnp.zeros((16,), jnp.float32)
        return carry

    lax.fori_loop(0, BGN // 16, zero, 0)

    def stage(s, carry):
        off = base + s * _SCH
        pltpu.sync_copy(pos_hbm.at[pl.ds(off, _SCH)], idx_v)
        pltpu.sync_copy(pr_hbm.at[pl.ds(off, _SCH)], p_v)

        def inner(i, c2):
            iv = idx_v[pl.ds(i * 16, 16)]
            pv = p_v[pl.ds(i * 16, 16)]
            plsc.addupdate_scatter(cbuf, [iv], pv)
            return c2

        lax.fori_loop(0, _SCH // 16, inner, 0)
        return carry

    lax.fori_loop(0, _N_SCH, stage, 0)
    pltpu.sync_copy(cbuf, out_hbm.at[wid])


def _sc_scatter(pos, pr):
    call = pl.kernel(
        _sc_scatter_body,
        mesh=plsc.VectorSubcoreMesh(core_axis_name="c", subcore_axis_name="s",
                                    num_cores=NC, num_subcores=NS),
        out_type=jax.ShapeDtypeStruct((NW, BGN), jnp.float32),
        scratch_types=[
            pltpu.VMEM((BGN,), jnp.float32),
            pltpu.VMEM((_SCH,), jnp.int32),
            pltpu.VMEM((_SCH,), jnp.float32),
        ],
        compiler_params=pltpu.CompilerParams(needs_layout_passes=False),
    )
    return call(pos, pr)


_TBN = 512


def _tc_t_body(q_ref, v_ref, x_ref, t_o):
    t_o[:, :CQ] = q_ref[0].T
    t_o[:, CQ:CQ + CV] = v_ref[0].T
    t_o[:, CQ + CV:CQ + CV + 3] = x_ref[0]


def _tc_t(q2, v2, xyz):
    grid = (B, N // _TBN)
    return pl.pallas_call(
        _tc_t_body,
        grid=grid,
        in_specs=[
            pl.BlockSpec((1, CQ, _TBN), lambda b, i: (b, 0, i)),
            pl.BlockSpec((1, CV, _TBN), lambda b, i: (b, 0, i)),
            pl.BlockSpec((1, _TBN, 3), lambda b, i: (b, i, 0)),
        ],
        out_specs=pl.BlockSpec((_TBN, CT), lambda b, i: (b * (N // _TBN) + i, 0)),
        out_shape=jax.ShapeDtypeStruct((B * N, CT), jnp.float32),
    )(q2, v2, xyz)


def _tc_a_body(gth, qt, xz, w, probs_o, lvv_o, y_o, st_o):
    Gt = gth[0]
    Q = qt[0][:, :CQ]
    X = xz[0]
    Kp = Gt[:, :CQ].reshape(BN_BLK, K, CQ)
    Vp = Gt[:, CQ:CQ + CV].reshape(BN_BLK, K, CV)
    xyzj = Gt[:, CQ + CV:CQ + CV + 3]

    for g in range(G):
        qg = Q[:, None, g * 64:(g + 1) * 64]
        la = (qg * Kp[..., g * 64:(g + 1) * 64]).sum(-1)
        la = la - la.max(-1, keepdims=True)
        e = jnp.exp(la)
        p = e / e.sum(-1, keepdims=True)
        probs_o[0, :, g * 16:(g + 1) * 16] = p
        lvv_o[0, :, g * 32:(g + 1) * 32] = (
            p[..., None] * Vp[..., g * 32:(g + 1) * 32]).sum(1)

    xi = jnp.broadcast_to(X[:, None, :], (BN_BLK, K, 3)).reshape(R_BLK, 3)
    rel = xi - xyzj
    dist = jnp.sqrt((rel * rel).sum(-1, keepdims=True))
    feats = jnp.concatenate([dist, rel, xi, xyzj], axis=-1)
    y = lax.dot_general(feats, w[...], (((1,), (1,)), ((), ())),
                        preferred_element_type=jnp.float32)
    y_o[0] = y

    ps = y.sum(0)
    psq = (y * y).sum(0)
    contrib = jnp.concatenate(
        [ps[None], psq[None], jnp.zeros((6, LPE_OUT), jnp.float32)], axis=0)
    first = (pl.program_id(0) == 0) & (pl.program_id(1) == 0)

    @pl.when(first)
    def _():
        st_o[...] = contrib

    @pl.when(jnp.logical_not(first))
    def _():
        st_o[...] = st_o[...] + contrib


def _tc_a(gathered, qT, xyz, lpe_w):
    grid = (B, N // BN_BLK)
    return pl.pallas_call(
        _tc_a_body,
        grid=grid,
        in_specs=[
            pl.BlockSpec((1, R_BLK, CT), lambda b, i: (b, i, 0)),
            pl.BlockSpec((1, BN_BLK, CT), lambda b, i: (b, i, 0)),
            pl.BlockSpec((1, BN_BLK, 3), lambda b, i: (b, i, 0)),
            pl.BlockSpec((LPE_OUT, 10), lambda b, i: (0, 0)),
        ],
        out_specs=[
            pl.BlockSpec((1, BN_BLK, G * K), lambda b, i: (b, i, 0)),
            pl.BlockSpec((1, BN_BLK, CV), lambda b, i: (b, i, 0)),
            pl.BlockSpec((1, R_BLK, LPE_OUT), lambda b, i: (b, i, 0)),
            pl.BlockSpec((8, LPE_OUT), lambda b, i: (0, 0)),
        ],
        out_shape=[
            jax.ShapeDtypeStruct((B, N, G * K), jnp.float32),
            jax.ShapeDtypeStruct((B, N, CV), jnp.float32),
            jax.ShapeDtypeStruct((B, N * K, LPE_OUT), jnp.float32),
            jax.ShapeDtypeStruct((8, LPE_OUT), jnp.float32),
        ],
    )(gathered, qT, xyz, lpe_w)


def _tc_b_body(y_ref, p_ref, lvv_ref, st_ref, gm_ref, bt_ref, f_o, lv_o):
    s = st_ref[0, :]
    sq = st_ref[1, :]
    cnt = float(TOT)
    mean = s / cnt
    var = sq / cnt - mean * mean
    inv = lax.rsqrt(var + 1e-5)
    gm = gm_ref[0, :]
    bt = bt_ref[0, :]
    y = y_ref[0]
    f = jnp.maximum((y - mean) * inv * gm + bt, 0.0)
    fT = f.T
    f_o[0] = jnp.broadcast_to(fT[None], (G, LPE_OUT, R_BLK))

    f3 = f.reshape(BN_BLK, K, LPE_OUT)
    P = p_ref[0]
    rows = []
    for g in range(G):
        pg = P[:, g * 16:(g + 1) * 16]
        lvf = (pg[:, :, None] * f3).sum(1)
        rows.append(lvv_ref[0][:, g * 32:(g + 1) * 32])
        rows.append(lvf)
    lv_rows = jnp.concatenate(rows, axis=-1)
    lv_o[0] = lv_rows.T


def _tc_b(y, probs, lvv, stats, gamma, beta):
    grid = (B, N // BN_BLK)
    return pl.pallas_call(
        _tc_b_body,
        grid=grid,
        in_specs=[
            pl.BlockSpec((1, R_BLK, LPE_OUT), lambda b, i: (b, i, 0)),
            pl.BlockSpec((1, BN_BLK, G * K), lambda b, i: (b, i, 0)),
            pl.BlockSpec((1, BN_BLK, CV), lambda b, i: (b, i, 0)),
            pl.BlockSpec((8, LPE_OUT), lambda b, i: (0, 0)),
            pl.BlockSpec((1, LPE_OUT), lambda b, i: (0, 0)),
            pl.BlockSpec((1, LPE_OUT), lambda b, i: (0, 0)),
        ],
        out_specs=[
            pl.BlockSpec((1, G, LPE_OUT, R_BLK), lambda b, i: (b, 0, 0, i)),
            pl.BlockSpec((1, CQ, BN_BLK), lambda b, i: (b, 0, i)),
        ],
        out_shape=[
            jax.ShapeDtypeStruct((B, G, LPE_OUT, N * K), jnp.float32),
            jax.ShapeDtypeStruct((B, CQ, N), jnp.float32),
        ],
    )(y, probs, lvv, stats, gamma, beta)


def _tc_d_body(x_ref, o_ref):
    o_ref[...] = x_ref[...].sum(0)


def _tc_d(partials):
    return pl.pallas_call(
        _tc_d_body,
        out_shape=jax.ShapeDtypeStruct((8, BGN // 8), jnp.float32),
    )(partials.reshape(NW, 8, BGN // 8))


def kernel(xyz, query, value, neigh_idx, idx_base, lpe_w, lpe_gamma, lpe_beta):
    q2 = query.reshape(B, CQ, N)
    v2 = value.reshape(B, CV, N)
    table = _tc_t(q2, v2, xyz)
    flat_idx = (neigh_idx + idx_base).reshape(-1).astype(jnp.int32)

    gathered = _sc_gather(table, flat_idx)

    probs, lvv, y, stats = _tc_a(
        gathered.reshape(B, N * K, CT), table.reshape(B, N, CT), xyz, lpe_w)

    f_out, lv_out = _tc_b(y, probs, lvv, stats,
                          lpe_gamma.reshape(1, LPE_OUT),
                          lpe_beta.reshape(1, LPE_OUT))

    bg = (jnp.arange(B, dtype=jnp.int32)[:, None, None, None] * G
          + jnp.arange(G, dtype=jnp.int32)[None, None, :, None])
    pos = (bg * N + neigh_idx[:, :, None, :]).reshape(-1)
    pr = probs.reshape(-1)
    partials = _sc_scatter(pos, pr)
    cent = _tc_d(partials).reshape(B, G, N)

    lv = lv_out.reshape(B, CQ, N, 1)
    f_xyz = f_out.reshape(B, G, LPE_OUT, N, K)
    return lv, f_xyz, cent


# speedup vs baseline: 1.4464x; 1.4464x over previous
import functools

import jax
import jax.numpy as jnp
from jax import lax
from jax.experimental import pallas as pl
from jax.experimental.pallas import tpu as pltpu
from jax.experimental.pallas import tpu_sc as plsc

B, N, K, G = 2, 4096, 16, 4
CQ, CV = 256, 128
LPE_OUT = 32
CT = 512
NC, NS = 2, 16
NW = NC * NS
TOT = B * N * K
BGN = B * G * N

BN_BLK = 128
R_BLK = BN_BLK * K


_ROWS_PER_W = TOT // NW
_GCH = 128
_N_GCH = _ROWS_PER_W // _GCH


def _sc_gather_body(table_hbm, idx_hbm, out_hbm, idx_v, rows_v, sem):
    wid = lax.axis_index("s") * NC + lax.axis_index("c")
    base = wid * _ROWS_PER_W

    def chunk(t, carry):
        off = base + t * _GCH
        pltpu.sync_copy(idx_hbm.at[pl.ds(off, _GCH)], idx_v)
        pltpu.async_copy(table_hbm.at[idx_v], rows_v, sem).wait()
        pltpu.sync_copy(rows_v, out_hbm.at[pl.ds(off, _GCH)])
        return carry

    lax.fori_loop(0, _N_GCH, chunk, 0)


def _sc_gather(table, flat_idx):
    call = pl.kernel(
        _sc_gather_body,
        mesh=plsc.VectorSubcoreMesh(core_axis_name="c", subcore_axis_name="s",
                                    num_cores=NC, num_subcores=NS),
        out_type=jax.ShapeDtypeStruct((TOT, CT), jnp.float32),
        scratch_types=[
            pltpu.VMEM((_GCH,), jnp.int32),
            pltpu.VMEM((_GCH, CT), jnp.float32),
            pltpu.SemaphoreType.DMA,
        ],
    )
    return call(table, flat_idx)


_ITEMS = B * N * G * K
_ITEMS_PER_W = _ITEMS // NW
_SCH = 2048
_N_SCH = _ITEMS_PER_W // _SCH


def _sc_scatter_body(pos_hbm, pr_hbm, out_hbm, cbuf, idx_v, p_v):
    wid = lax.axis_index("s") * NC + lax.axis_index("c")
    base = wid * _ITEMS_PER_W

    def zero(i, carry):
        cbuf[pl.ds(i * 16, 16)] = jnp.zeros((16,), jnp.float32)
        return carry

    lax.fori_loop(0, BGN // 16, zero, 0)

    def stage(s, carry):
        off = base + s * _SCH
        pltpu.sync_copy(pos_hbm.at[pl.ds(off, _SCH)], idx_v)
        pltpu.sync_copy(pr_hbm.at[pl.ds(off, _SCH)], p_v)

        def inner(i, c2):
            iv = idx_v[pl.ds(i * 16, 16)]
            pv = p_v[pl.ds(i * 16, 16)]
            plsc.addupdate_scatter(cbuf, [iv], pv)
            return c2

        lax.fori_loop(0, _SCH // 16, inner, 0)
        return carry

    lax.fori_loop(0, _N_SCH, stage, 0)
    pltpu.sync_copy(cbuf, out_hbm.at[wid])


def _sc_scatter(pos, pr):
    call = pl.kernel(
        _sc_scatter_body,
        mesh=plsc.VectorSubcoreMesh(core_axis_name="c", subcore_axis_name="s",
                                    num_cores=NC, num_subcores=NS),
        out_type=jax.ShapeDtypeStruct((NW, BGN), jnp.float32),
        scratch_types=[
            pltpu.VMEM((BGN,), jnp.float32),
            pltpu.VMEM((_SCH,), jnp.int32),
            pltpu.VMEM((_SCH,), jnp.float32),
        ],
        compiler_params=pltpu.CompilerParams(needs_layout_passes=False),
    )
    return call(pos, pr)


_TBN = 512


def _tc_t_body(q_ref, v_ref, x_ref, t_o):
    t_o[:, :CQ] = q_ref[0].T
    t_o[:, CQ:CQ + CV] = v_ref[0].T
    t_o[:, CQ + CV:CQ + CV + 3] = x_ref[0]


def _tc_t(q2, v2, xyz):
    grid = (B, N // _TBN)
    return pl.pallas_call(
        _tc_t_body,
        grid=grid,
        in_specs=[
            pl.BlockSpec((1, CQ, _TBN), lambda b, i: (b, 0, i)),
            pl.BlockSpec((1, CV, _TBN), lambda b, i: (b, 0, i)),
            pl.BlockSpec((1, _TBN, 3), lambda b, i: (b, i, 0)),
        ],
        out_specs=pl.BlockSpec((_TBN, CT), lambda b, i: (b * (N // _TBN) + i, 0)),
        out_shape=jax.ShapeDtypeStruct((B * N, CT), jnp.float32),
    )(q2, v2, xyz)


def _tc_a_body(gth, qt, xz, w, probs_o, lvv_o, y_o, st_o):
    Gt = gth[0]
    Q = qt[0][:, :CQ]
    X = xz[0]
    Kp = Gt[:, :CQ].reshape(BN_BLK, K, CQ)
    Vp = Gt[:, CQ:CQ + CV].reshape(BN_BLK, K, CV)
    xyzj = Gt[:, CQ + CV:CQ + CV + 3]

    for g in range(G):
        qg = Q[:, None, g * 64:(g + 1) * 64]
        la = (qg * Kp[..., g * 64:(g + 1) * 64]).sum(-1)
        la = la - la.max(-1, keepdims=True)
        e = jnp.exp(la)
        p = e / e.sum(-1, keepdims=True)
        probs_o[0, :, g * 16:(g + 1) * 16] = p
        lvv_o[0, :, g * 32:(g + 1) * 32] = (
            p[..., None] * Vp[..., g * 32:(g + 1) * 32]).sum(1)

    xi = jnp.broadcast_to(X[:, None, :], (BN_BLK, K, 3)).reshape(R_BLK, 3)
    rel = xi - xyzj
    dist = jnp.sqrt((rel * rel).sum(-1, keepdims=True))
    feats = jnp.concatenate([dist, rel, xi, xyzj], axis=-1)
    y = lax.dot_general(feats, w[...], (((1,), (1,)), ((), ())),
                        preferred_element_type=jnp.float32)
    y_o[0] = y

    ps = y.sum(0)
    psq = (y * y).sum(0)
    contrib = jnp.concatenate(
        [ps[None], psq[None], jnp.zeros((6, LPE_OUT), jnp.float32)], axis=0)
    first = (pl.program_id(0) == 0) & (pl.program_id(1) == 0)

    @pl.when(first)
    def _():
        st_o[...] = contrib

    @pl.when(jnp.logical_not(first))
    def _():
        st_o[...] = st_o[...] + contrib


def _tc_a(gathered, qT, xyz, lpe_w):
    grid = (B, N // BN_BLK)
    return pl.pallas_call(
        _tc_a_body,
        grid=grid,
        in_specs=[
            pl.BlockSpec((1, R_BLK, CT), lambda b, i: (b, i, 0)),
            pl.BlockSpec((1, BN_BLK, CT), lambda b, i: (b, i, 0)),
            pl.BlockSpec((1, BN_BLK, 3), lambda b, i: (b, i, 0)),
            pl.BlockSpec((LPE_OUT, 10), lambda b, i: (0, 0)),
        ],
        out_specs=[
            pl.BlockSpec((1, BN_BLK, G * K), lambda b, i: (b, i, 0)),
            pl.BlockSpec((1, BN_BLK, CV), lambda b, i: (b, i, 0)),
            pl.BlockSpec((1, R_BLK, LPE_OUT), lambda b, i: (b, i, 0)),
            pl.BlockSpec((8, LPE_OUT), lambda b, i: (0, 0)),
        ],
        out_shape=[
            jax.ShapeDtypeStruct((B, N, G * K), jnp.float32),
            jax.ShapeDtypeStruct((B, N, CV), jnp.float32),
            jax.ShapeDtypeStruct((B, N * K, LPE_OUT), jnp.float32),
            jax.ShapeDtypeStruct((8, LPE_OUT), jnp.float32),
        ],
    )(gathered, qT, xyz, lpe_w)


def _tc_b_body(y_ref, p_ref, lvv_ref, st_ref, gm_ref, bt_ref, f_o, lv_o):
    s = st_ref[0, :]
    sq = st_ref[1, :]
    cnt = float(TOT)
    mean = s / cnt
    var = sq / cnt - mean * mean
    inv = lax.rsqrt(var + 1e-5)
    gm = gm_ref[0, :]
    bt = bt_ref[0, :]
    y = y_ref[0]
    f = jnp.maximum((y - mean) * inv * gm + bt, 0.0)
    f3 = f.reshape(BN_BLK, K, LPE_OUT)
    for j in range(K):
        fj = f3[:, j, :].T
        f_o[0, :, :, j, :] = jnp.broadcast_to(fj[None], (G, LPE_OUT, BN_BLK))

    P = p_ref[0]
    rows = []
    for g in range(G):
        pg = P[:, g * 16:(g + 1) * 16]
        lvf = (pg[:, :, None] * f3).sum(1)
        rows.append(lvv_ref[0][:, g * 32:(g + 1) * 32])
        rows.append(lvf)
    lv_rows = jnp.concatenate(rows, axis=-1)
    lv_o[0] = lv_rows.T


def _tc_b(y, probs, lvv, stats, gamma, beta):
    grid = (B, N // BN_BLK)
    return pl.pallas_call(
        _tc_b_body,
        grid=grid,
        in_specs=[
            pl.BlockSpec((1, R_BLK, LPE_OUT), lambda b, i: (b, i, 0)),
            pl.BlockSpec((1, BN_BLK, G * K), lambda b, i: (b, i, 0)),
            pl.BlockSpec((1, BN_BLK, CV), lambda b, i: (b, i, 0)),
            pl.BlockSpec((8, LPE_OUT), lambda b, i: (0, 0)),
            pl.BlockSpec((1, LPE_OUT), lambda b, i: (0, 0)),
            pl.BlockSpec((1, LPE_OUT), lambda b, i: (0, 0)),
        ],
        out_specs=[
            pl.BlockSpec((1, G, LPE_OUT, K, BN_BLK), lambda b, i: (b, 0, 0, 0, i)),
            pl.BlockSpec((1, CQ, BN_BLK), lambda b, i: (b, 0, i)),
        ],
        out_shape=[
            jax.ShapeDtypeStruct((B, G, LPE_OUT, K, N), jnp.float32),
            jax.ShapeDtypeStruct((B, CQ, N), jnp.float32),
        ],
    )(y, probs, lvv, stats, gamma, beta)


def _tc_d_body(x_ref, o_ref):
    o_ref[...] = x_ref[...].sum(0)


def _tc_d(partials):
    return pl.pallas_call(
        _tc_d_body,
        out_shape=jax.ShapeDtypeStruct((8, BGN // 8), jnp.float32),
    )(partials.reshape(NW, 8, BGN // 8))


def kernel(xyz, query, value, neigh_idx, idx_base, lpe_w, lpe_gamma, lpe_beta):
    q2 = query.reshape(B, CQ, N)
    v2 = value.reshape(B, CV, N)
    table = _tc_t(q2, v2, xyz)
    flat_idx = (neigh_idx + idx_base).reshape(-1).astype(jnp.int32)

    gathered = _sc_gather(table, flat_idx)

    probs, lvv, y, stats = _tc_a(
        gathered.reshape(B, N * K, CT), table.reshape(B, N, CT), xyz, lpe_w)

    f_out, lv_out = _tc_b(y, probs, lvv, stats,
                          lpe_gamma.reshape(1, LPE_OUT),
                          lpe_beta.reshape(1, LPE_OUT))

    bg = (jnp.arange(B, dtype=jnp.int32)[:, None, None, None] * G
          + jnp.arange(G, dtype=jnp.int32)[None, None, :, None])
    pos = (bg * N + neigh_idx[:, :, None, :]).reshape(-1)
    pr = probs.reshape(-1)
    partials = _sc_scatter(pos, pr)
    cent = _tc_d(partials).reshape(B, G, N)

    lv = lv_out.reshape(B, CQ, N, 1)
    f_xyz = jnp.transpose(f_out, (0, 1, 2, 4, 3))
    return lv, f_xyz, cent


# speedup vs baseline: 1.5009x; 1.0377x over previous
import functools

import jax
import jax.numpy as jnp
from jax import lax
from jax.experimental import pallas as pl
from jax.experimental.pallas import tpu as pltpu
from jax.experimental.pallas import tpu_sc as plsc

B, N, K, G = 2, 4096, 16, 4
CQ, CV = 256, 128
LPE_OUT = 32
CT = 512
NC, NS = 2, 16
NW = NC * NS
TOT = B * N * K
BGN = B * G * N

BN_BLK = 128
R_BLK = BN_BLK * K


_ROWS_PER_W = TOT // NW
_GCH = 64
_N_GCH = _ROWS_PER_W // _GCH


def _sc_gather_body(table_hbm, idx_hbm, out_hbm, idx_v, rows_v,
                    sg0, sg1, sw0, sw1):
    wid = lax.axis_index("s") * NC + lax.axis_index("c")
    base = wid * _ROWS_PER_W
    sg = [sg0, sg1]
    sw = [sw0, sw1]

    def start_gather(t, s):
        off = base + t * _GCH
        pltpu.sync_copy(idx_hbm.at[pl.ds(off, _GCH)], idx_v.at[s])
        return pltpu.async_copy(table_hbm.at[idx_v.at[s]], rows_v.at[s], sg[s])

    gat = [None, None]
    out = [None, None]
    gat[0] = start_gather(0, 0)
    for t in range(_N_GCH):
        s = t % 2
        ns = (t + 1) % 2
        if t + 1 < _N_GCH:
            if out[ns] is not None:
                out[ns].wait()
            gat[ns] = start_gather(t + 1, ns)
        gat[s].wait()
        out[s] = pltpu.async_copy(
            rows_v.at[s], out_hbm.at[pl.ds(base + t * _GCH, _GCH)], sw[s])
    out[0].wait()
    out[1].wait()


def _sc_gather(table, flat_idx):
    call = pl.kernel(
        _sc_gather_body,
        mesh=plsc.VectorSubcoreMesh(core_axis_name="c", subcore_axis_name="s",
                                    num_cores=NC, num_subcores=NS),
        out_type=jax.ShapeDtypeStruct((TOT, CT), jnp.float32),
        scratch_types=[
            pltpu.VMEM((2, _GCH), jnp.int32),
            pltpu.VMEM((2, _GCH, CT), jnp.float32),
            pltpu.SemaphoreType.DMA,
            pltpu.SemaphoreType.DMA,
            pltpu.SemaphoreType.DMA,
            pltpu.SemaphoreType.DMA,
        ],
    )
    return call(table, flat_idx)


_ITEMS = B * N * G * K
_ITEMS_PER_W = _ITEMS // NW
_SCH = 2048
_N_SCH = _ITEMS_PER_W // _SCH


def _sc_scatter_body(pos_hbm, pr_hbm, out_hbm, cbuf, idx_v, p_v):
    wid = lax.axis_index("s") * NC + lax.axis_index("c")
    base = wid * _ITEMS_PER_W

    def zero(i, carry):
        cbuf[pl.ds(i * 16, 16)] = jnp.zeros((16,), jnp.float32)
        return carry

    lax.fori_loop(0, BGN // 16, zero, 0)

    def stage(s, carry):
        off = base + s * _SCH
        pltpu.sync_copy(pos_hbm.at[pl.ds(off, _SCH)], idx_v)
        pltpu.sync_copy(pr_hbm.at[pl.ds(off, _SCH)], p_v)

        def inner(i, c2):
            iv = idx_v[pl.ds(i * 16, 16)]
            pv = p_v[pl.ds(i * 16, 16)]
            plsc.addupdate_scatter(cbuf, [iv], pv)
            return c2

        lax.fori_loop(0, _SCH // 16, inner, 0)
        return carry

    lax.fori_loop(0, _N_SCH, stage, 0)
    pltpu.sync_copy(cbuf, out_hbm.at[wid])


def _sc_scatter(pos, pr):
    call = pl.kernel(
        _sc_scatter_body,
        mesh=plsc.VectorSubcoreMesh(core_axis_name="c", subcore_axis_name="s",
                                    num_cores=NC, num_subcores=NS),
        out_type=jax.ShapeDtypeStruct((NW, BGN), jnp.float32),
        scratch_types=[
            pltpu.VMEM((BGN,), jnp.float32),
            pltpu.VMEM((_SCH,), jnp.int32),
            pltpu.VMEM((_SCH,), jnp.float32),
        ],
        compiler_params=pltpu.CompilerParams(needs_layout_passes=False),
    )
    return call(pos, pr)


_TBN = 512


def _tc_t_body(q_ref, v_ref, x_ref, t_o):
    t_o[:, :CQ] = q_ref[0].T
    t_o[:, CQ:CQ + CV] = v_ref[0].T
    t_o[:, CQ + CV:CQ + CV + 3] = x_ref[0]


def _tc_t(q2, v2, xyz):
    grid = (B, N // _TBN)
    return pl.pallas_call(
        _tc_t_body,
        grid=grid,
        in_specs=[
            pl.BlockSpec((1, CQ, _TBN), lambda b, i: (b, 0, i)),
            pl.BlockSpec((1, CV, _TBN), lambda b, i: (b, 0, i)),
            pl.BlockSpec((1, _TBN, 3), lambda b, i: (b, i, 0)),
        ],
        out_specs=pl.BlockSpec((_TBN, CT), lambda b, i: (b * (N // _TBN) + i, 0)),
        out_shape=jax.ShapeDtypeStruct((B * N, CT), jnp.float32),
    )(q2, v2, xyz)


def _tc_a_body(gth, qt, xz, w, probs_o, lvv_o, y_o, st_o):
    Gt = gth[0]
    Q = qt[0][:, :CQ]
    X = xz[0]
    Kp = Gt[:, :CQ].reshape(BN_BLK, K, CQ)
    Vp = Gt[:, CQ:CQ + CV].reshape(BN_BLK, K, CV)
    xyzj = Gt[:, CQ + CV:CQ + CV + 3]

    for g in range(G):
        qg = Q[:, None, g * 64:(g + 1) * 64]
        la = (qg * Kp[..., g * 64:(g + 1) * 64]).sum(-1)
        la = la - la.max(-1, keepdims=True)
        e = jnp.exp(la)
        p = e / e.sum(-1, keepdims=True)
        probs_o[0, :, g * 16:(g + 1) * 16] = p
        lvv_o[0, :, g * 32:(g + 1) * 32] = (
            p[..., None] * Vp[..., g * 32:(g + 1) * 32]).sum(1)

    xi = jnp.broadcast_to(X[:, None, :], (BN_BLK, K, 3)).reshape(R_BLK, 3)
    rel = xi - xyzj
    dist = jnp.sqrt((rel * rel).sum(-1, keepdims=True))
    feats = jnp.concatenate([dist, rel, xi, xyzj], axis=-1)
    y = lax.dot_general(feats, w[...], (((1,), (1,)), ((), ())),
                        preferred_element_type=jnp.float32)
    y_o[0] = y

    ps = y.sum(0)
    psq = (y * y).sum(0)
    contrib = jnp.concatenate(
        [ps[None], psq[None], jnp.zeros((6, LPE_OUT), jnp.float32)], axis=0)
    first = (pl.program_id(0) == 0) & (pl.program_id(1) == 0)

    @pl.when(first)
    def _():
        st_o[...] = contrib

    @pl.when(jnp.logical_not(first))
    def _():
        st_o[...] = st_o[...] + contrib


def _tc_a(gathered, qT, xyz, lpe_w):
    grid = (B, N // BN_BLK)
    return pl.pallas_call(
        _tc_a_body,
        grid=grid,
        in_specs=[
            pl.BlockSpec((1, R_BLK, CT), lambda b, i: (b, i, 0)),
            pl.BlockSpec((1, BN_BLK, CT), lambda b, i: (b, i, 0)),
            pl.BlockSpec((1, BN_BLK, 3), lambda b, i: (b, i, 0)),
            pl.BlockSpec((LPE_OUT, 10), lambda b, i: (0, 0)),
        ],
        out_specs=[
            pl.BlockSpec((1, BN_BLK, G * K), lambda b, i: (b, i, 0)),
            pl.BlockSpec((1, BN_BLK, CV), lambda b, i: (b, i, 0)),
            pl.BlockSpec((1, R_BLK, LPE_OUT), lambda b, i: (b, i, 0)),
            pl.BlockSpec((8, LPE_OUT), lambda b, i: (0, 0)),
        ],
        out_shape=[
            jax.ShapeDtypeStruct((B, N, G * K), jnp.float32),
            jax.ShapeDtypeStruct((B, N, CV), jnp.float32),
            jax.ShapeDtypeStruct((B, N * K, LPE_OUT), jnp.float32),
            jax.ShapeDtypeStruct((8, LPE_OUT), jnp.float32),
        ],
    )(gathered, qT, xyz, lpe_w)


def _tc_b_body(y_ref, p_ref, lvv_ref, st_ref, gm_ref, bt_ref, f_o, lv_o):
    s = st_ref[0, :]
    sq = st_ref[1, :]
    cnt = float(TOT)
    mean = s / cnt
    var = sq / cnt - mean * mean
    inv = lax.rsqrt(var + 1e-5)
    gm = gm_ref[0, :]
    bt = bt_ref[0, :]
    y = y_ref[0]
    f = jnp.maximum((y - mean) * inv * gm + bt, 0.0)
    f3 = f.reshape(BN_BLK, K, LPE_OUT)
    for j in range(K):
        fj = f3[:, j, :].T
        f_o[0, :, :, j, :] = jnp.broadcast_to(fj[None], (G, LPE_OUT, BN_BLK))

    P = p_ref[0]
    rows = []
    for g in range(G):
        pg = P[:, g * 16:(g + 1) * 16]
        lvf = (pg[:, :, None] * f3).sum(1)
        rows.append(lvv_ref[0][:, g * 32:(g + 1) * 32])
        rows.append(lvf)
    lv_rows = jnp.concatenate(rows, axis=-1)
    lv_o[0] = lv_rows.T


def _tc_b(y, probs, lvv, stats, gamma, beta):
    grid = (B, N // BN_BLK)
    return pl.pallas_call(
        _tc_b_body,
        grid=grid,
        in_specs=[
            pl.BlockSpec((1, R_BLK, LPE_OUT), lambda b, i: (b, i, 0)),
            pl.BlockSpec((1, BN_BLK, G * K), lambda b, i: (b, i, 0)),
            pl.BlockSpec((1, BN_BLK, CV), lambda b, i: (b, i, 0)),
            pl.BlockSpec((8, LPE_OUT), lambda b, i: (0, 0)),
            pl.BlockSpec((1, LPE_OUT), lambda b, i: (0, 0)),
            pl.BlockSpec((1, LPE_OUT), lambda b, i: (0, 0)),
        ],
        out_specs=[
            pl.BlockSpec((1, G, LPE_OUT, K, BN_BLK), lambda b, i: (b, 0, 0, 0, i)),
            pl.BlockSpec((1, CQ, BN_BLK), lambda b, i: (b, 0, i)),
        ],
        out_shape=[
            jax.ShapeDtypeStruct((B, G, LPE_OUT, K, N), jnp.float32),
            jax.ShapeDtypeStruct((B, CQ, N), jnp.float32),
        ],
    )(y, probs, lvv, stats, gamma, beta)


def _tc_d_body(x_ref, o_ref):
    o_ref[...] = x_ref[...].sum(0)


def _tc_d(partials):
    return pl.pallas_call(
        _tc_d_body,
        out_shape=jax.ShapeDtypeStruct((8, BGN // 8), jnp.float32),
    )(partials.reshape(NW, 8, BGN // 8))


def kernel(xyz, query, value, neigh_idx, idx_base, lpe_w, lpe_gamma, lpe_beta):
    q2 = query.reshape(B, CQ, N)
    v2 = value.reshape(B, CV, N)
    table = _tc_t(q2, v2, xyz)
    flat_idx = (neigh_idx + idx_base).reshape(-1).astype(jnp.int32)

    gathered = _sc_gather(table, flat_idx)

    probs, lvv, y, stats = _tc_a(
        gathered.reshape(B, N * K, CT), table.reshape(B, N, CT), xyz, lpe_w)

    f_out, lv_out = _tc_b(y, probs, lvv, stats,
                          lpe_gamma.reshape(1, LPE_OUT),
                          lpe_beta.reshape(1, LPE_OUT))

    bg = (jnp.arange(B, dtype=jnp.int32)[:, None, None, None] * G
          + jnp.arange(G, dtype=jnp.int32)[None, None, :, None])
    pos = (bg * N + neigh_idx[:, :, None, :]).reshape(-1)
    pr = probs.reshape(-1)
    partials = _sc_scatter(pos, pr)
    cent = _tc_d(partials).reshape(B, G, N)

    lv = lv_out.reshape(B, CQ, N, 1)
    f_xyz = jnp.transpose(f_out, (0, 1, 2, 4, 3))
    return lv, f_xyz, cent
